# Initial kernel scaffold; baseline (speedup 1.0000x reference)
import jax
import jax.numpy as jnp
from jax.experimental import pallas as pl


def kernel(cost, spg):
    raise NotImplementedError("write your pallas kernel here")


# speedup vs baseline: 3.1910x; 3.1910x over previous
import functools

import jax
import jax.numpy as jnp
from jax import lax
from jax.experimental import pallas as pl
from jax.experimental.pallas import tpu as pltpu
from jax.experimental.pallas import tpu_sc as plsc

B, D, H, W = 4, 48, 128, 256
UP = 4
HO, WO = H * UP, W * UP
BANDS = 8
RPB = H // BANDS
SLAB = RPB + 2
DCH = 6
NCH = D // DCH
UW = WO + 16
ORPB = RPB * UP
L = 16


def _top2_update(v, m1, m2, i1, i2, dscaled):
    dv = jnp.full((L,), dscaled, jnp.float32)
    c1 = v > m1
    c2 = v > m2
    m2n = jnp.where(c1, m1, jnp.where(c2, v, m2))
    i2n = jnp.where(c1, i1, jnp.where(c2, dv, i2))
    m1n = jnp.where(c1, v, m1)
    i1n = jnp.where(c1, dv, i1)
    return m1n, m2n, i1n, i2n


def _sc_body(cost_hbm, spg_hbm, out_hbm,
             costbuf, m1s, m2s, i1s, i2s, d4p, u, spgbuf, outbuf,
             sem_c0, sem_c1, sem_s0, sem_s1, sem_o0, sem_o1):
    cid = lax.axis_index("c")
    sid = lax.axis_index("s")
    wid = sid * 2 + cid
    b = wid // BANDS
    band = wid % BANDS
    y0 = band * RPB
    rstart = jnp.clip(y0 - 1, 0, H - SLAB)
    gofs = (y0 - 1) - rstart
    ybase = band * ORPB

    sem_c = (sem_c0, sem_c1)
    sem_s = (sem_s0, sem_s1)
    sem_o = (sem_o0, sem_o1)

    def cost_copy(ci, par):
        return pltpu.make_async_copy(
            cost_hbm.at[b, pl.ds(ci * DCH, DCH), pl.ds(rstart, SLAB), :],
            costbuf.at[par], sem_c[par])

    def spg_copy(row, par):
        return pltpu.make_async_copy(
            spg_hbm.at[b, :, ybase + row, :], spgbuf.at[par], sem_s[par])

    def out_copy(row, par):
        return pltpu.make_async_copy(
            outbuf.at[par], out_hbm.at[b, ybase + row, :], sem_o[par])

    cost_copy(0, 0).start()
    spg_copy(jnp.int32(0), 0).start()
    spg_copy(jnp.int32(1), 1).start()

    neg = jnp.full((L,), -jnp.inf, jnp.float32)
    zero = jnp.zeros((L,), jnp.float32)

    @pl.loop(0, SLAB)
    def _(r):
        for jj in range(W // L):
            sl = pl.ds(jj * L, L)
            m1s[r, sl] = neg
            m2s[r, sl] = neg
            i1s[r, sl] = zero
            i2s[r, sl] = zero

    for ci in range(NCH):
        par = ci & 1
        if ci + 1 < NCH:
            cost_copy(ci + 1, 1 - par).start()
        cost_copy(ci, par).wait()

        @pl.loop(0, SLAB)
        def _(r, ci=ci, par=par):
            @pl.loop(0, W // (2 * L))
            def _(jj, r=r, ci=ci, par=par):
                for half in range(2):
                    sl = pl.ds(jj * L + half * (W // 2), L)
                    m1 = m1s[r, sl]
                    m2 = m2s[r, sl]
                    i1 = i1s[r, sl]
                    i2 = i2s[r, sl]
                    for dd in range(DCH):
                        v = costbuf[par, dd, r, sl]
                        m1, m2, i1, i2 = _top2_update(
                            v, m1, m2, i1, i2, float(4 * (ci * DCH + dd)))
                    m1s[r, sl] = m1
                    m2s[r, sl] = m2
                    i1s[r, sl] = i1
                    i2s[r, sl] = i2

    @pl.loop(0, SLAB)
    def _(r):
        ro = jnp.clip(r + gofs, 0, SLAB - 1)
        grow = y0 - 1 + r
        validf = ((grow >= 0) & (grow < H)).astype(jnp.float32)
        for jj in range(W // L):
            sl = pl.ds(jj * L, L)
            m1 = m1s[ro, sl]
            m2 = m2s[ro, sl]
            i1 = i1s[ro, sl]
            i2 = i2s[ro, sl]
            e = jnp.exp(m2 - m1)
            w1 = 1.0 / (1.0 + e)
            d4 = (i2 + (i1 - i2) * w1) * validf
            d4p[r, sl] = d4

    q = lax.iota(jnp.int32, L) >> 2
    zi = jnp.zeros((L,), jnp.int32)

    @pl.loop(0, SLAB)
    def _(r):
        rv = zi + r

        @pl.loop(0, UW // L)
        def _(j, rv=rv):
            c = q + 4 * j
            inb = (c >= 1) & (c <= W)
            idx = jnp.clip(c - 1, 0, W - 1)
            g = plsc.load_gather(d4p, [rv, idx])
            u[r, pl.ds(j * L, L)] = jnp.where(inb, g, 0.0)

    @pl.loop(0, ORPB // 2)
    def _(t):
        for p in range(2):
            row = 2 * t + p
            spg_copy(row, p).wait()

            @pl.when(t > 0)
            def _(p=p):
                out_copy(jnp.int32(0), p).wait()

            ly = row >> 2

            @pl.loop(0, WO // L, unroll=2)
            def _(j, p=p, ly=ly):
                acc = None
                for ky in range(3):
                    for kx in range(3):
                        sv = spgbuf[p, 3 * ky + kx, pl.ds(j * L, L)]
                        uv = u[ly + ky, pl.ds(j * L + 4 * kx, L)]
                        term = sv * uv
                        acc = term if acc is None else acc + term
                outbuf[p, pl.ds(j * L, L)] = acc

            out_copy(row, p).start()
            nrow = jnp.minimum(row + 2, ORPB - 1)
            spg_copy(nrow, p).start()

    for p in range(2):
        spg_copy(jnp.int32(0), p).wait()
        out_copy(jnp.int32(0), p).wait()


@functools.cache
def _sc_regression():
    return functools.partial(
        pl.kernel,
        out_type=jax.ShapeDtypeStruct((B, HO, WO), jnp.float32),
        mesh=plsc.VectorSubcoreMesh(core_axis_name="c", subcore_axis_name="s"),
        compiler_params=pltpu.CompilerParams(use_tc_tiling_on_sc=False, needs_layout_passes=False),
        scratch_types=[
            pltpu.VMEM((2, DCH, SLAB, W), jnp.float32),
            pltpu.VMEM((SLAB, W), jnp.float32),
            pltpu.VMEM((SLAB, W), jnp.float32),
            pltpu.VMEM((SLAB, W), jnp.float32),
            pltpu.VMEM((SLAB, W), jnp.float32),
            pltpu.VMEM((SLAB, W), jnp.float32),
            pltpu.VMEM((SLAB, UW), jnp.float32),
            pltpu.VMEM((2, 9, WO), jnp.float32),
            pltpu.VMEM((2, WO), jnp.float32),
            pltpu.SemaphoreType.DMA,
            pltpu.SemaphoreType.DMA,
            pltpu.SemaphoreType.DMA,
            pltpu.SemaphoreType.DMA,
            pltpu.SemaphoreType.DMA,
            pltpu.SemaphoreType.DMA,
        ],
    )(_sc_body)


@jax.jit
def kernel(cost, spg):
    return _sc_regression()(cost.reshape(B, D, H, W), spg)


# speedup vs baseline: 3.3165x; 1.0393x over previous
import functools

import jax
import jax.numpy as jnp
from jax import lax
from jax.experimental import pallas as pl
from jax.experimental.pallas import tpu as pltpu
from jax.experimental.pallas import tpu_sc as plsc

B, D, H, W = 4, 48, 128, 256
UP = 4
HO, WO = H * UP, W * UP
BANDS = 8
RPB = H // BANDS
SLAB = RPB + 2
DCH = 2
NCH = D // DCH
WP = W + 8
ORPB = RPB * UP
NG = RPB
L = 16


def _top2_update(v, m1, m2, i1, i2, dscaled):
    dv = jnp.full((L,), dscaled, jnp.float32)
    c1 = v > m1
    c2 = v > m2
    m2n = jnp.where(c1, m1, jnp.where(c2, v, m2))
    i2n = jnp.where(c1, i1, jnp.where(c2, dv, i2))
    m1n = jnp.where(c1, v, m1)
    i1n = jnp.where(c1, dv, i1)
    return m1n, m2n, i1n, i2n


def _sc_body(cost_hbm, spg_hbm, out_hbm,
             costbuf, m1s, m2s, i1s, i2s, d4p, spgbuf, outbuf,
             sem_c0, sem_c1, sem_s0, sem_s1, sem_o0, sem_o1):
    cid = lax.axis_index("c")
    sid = lax.axis_index("s")
    wid = sid * 2 + cid
    b = wid // BANDS
    band = wid % BANDS
    y0 = band * RPB
    rstart = jnp.clip(y0 - 1, 0, H - SLAB)
    gofs = (y0 - 1) - rstart
    ybase = band * ORPB

    sem_c = (sem_c0, sem_c1)
    sem_s = (sem_s0, sem_s1)
    sem_o = (sem_o0, sem_o1)

    def cost_copy(ci, par):
        return pltpu.make_async_copy(
            cost_hbm.at[b, pl.ds(ci * DCH, DCH), pl.ds(rstart, SLAB), :],
            costbuf.at[par], sem_c[par])

    def spg_copy(g, par):
        return pltpu.make_async_copy(
            spg_hbm.at[b, :, pl.ds(ybase + 4 * g, 4), :],
            spgbuf.at[par], sem_s[par])

    def out_copy(g, par):
        return pltpu.make_async_copy(
            outbuf.at[par], out_hbm.at[b, pl.ds(ybase + 4 * g, 4), :],
            sem_o[par])

    cost_copy(0, 0).start()
    spg_copy(jnp.int32(0), 0).start()
    spg_copy(jnp.int32(1), 1).start()

    neg = jnp.full((L,), -jnp.inf, jnp.float32)
    zero = jnp.zeros((L,), jnp.float32)

    @pl.loop(0, SLAB)
    def _(r):
        for jj in range(W // L):
            sl = pl.ds(jj * L, L)
            m1s[r, sl] = neg
            m2s[r, sl] = neg
            i1s[r, sl] = zero
            i2s[r, sl] = zero
        d4p[r, pl.ds(0, L)] = zero
        d4p[r, pl.ds(WP - L, L)] = zero

    for ci in range(NCH):
        par = ci & 1
        if ci + 1 < NCH:
            cost_copy(ci + 1, 1 - par).start()
        cost_copy(ci, par).wait()

        @pl.loop(0, SLAB)
        def _(r, ci=ci, par=par):
            @pl.loop(0, W // (2 * L))
            def _(jj, r=r, ci=ci, par=par):
                for half in range(2):
                    sl = pl.ds(jj * L + half * (W // 2), L)
                    m1 = m1s[r, sl]
                    m2 = m2s[r, sl]
                    i1 = i1s[r, sl]
                    i2 = i2s[r, sl]
                    for dd in range(DCH):
                        v = costbuf[par, dd, r, sl]
                        m1, m2, i1, i2 = _top2_update(
                            v, m1, m2, i1, i2, float(4 * (ci * DCH + dd)))
                    m1s[r, sl] = m1
                    m2s[r, sl] = m2
                    i1s[r, sl] = i1
                    i2s[r, sl] = i2

    @pl.loop(0, SLAB)
    def _(r):
        ro = jnp.clip(r + gofs, 0, SLAB - 1)
        grow = y0 - 1 + r
        validf = ((grow >= 0) & (grow < H)).astype(jnp.float32)
        for jj in range(W // L):
            sl = pl.ds(jj * L, L)
            m1 = m1s[ro, sl]
            m2 = m2s[ro, sl]
            i1 = i1s[ro, sl]
            i2 = i2s[ro, sl]
            e = jnp.exp(m2 - m1)
            w1 = 1.0 / (1.0 + e)
            d4 = (i2 + (i1 - i2) * w1) * validf
            d4p[r, pl.ds(1 + jj * L, L)] = d4

    q = lax.iota(jnp.int32, L) >> 2
    zi = jnp.zeros((L,), jnp.int32)

    @pl.loop(0, NG // 2)
    def _(t):
        for p in range(2):
            g = 2 * t + p
            spg_copy(g, p).wait()

            @pl.when(t > 0)
            def _(p=p):
                out_copy(jnp.int32(0), p).wait()

            rv = [zi + (g + ky) for ky in range(3)]

            @pl.loop(0, WO // L)
            def _(j, p=p, rv=rv):
                c0 = q + 4 * j
                taps = [
                    plsc.load_gather(d4p, [rv[ky], c0 + kx])
                    for ky in range(3) for kx in range(3)
                ]
                for r4 in range(4):
                    acc = None
                    for ti in range(9):
                        term = taps[ti] * spgbuf[p, ti, r4, pl.ds(j * L, L)]
                        acc = term if acc is None else acc + term
                    outbuf[p, r4, pl.ds(j * L, L)] = acc

            out_copy(g, p).start()
            spg_copy(jnp.minimum(g + 2, NG - 1), p).start()

    for p in range(2):
        spg_copy(jnp.int32(0), p).wait()
        out_copy(jnp.int32(0), p).wait()


@functools.cache
def _sc_regression():
    return functools.partial(
        pl.kernel,
        out_type=jax.ShapeDtypeStruct((B, HO, WO), jnp.float32),
        mesh=plsc.VectorSubcoreMesh(core_axis_name="c", subcore_axis_name="s"),
        compiler_params=pltpu.CompilerParams(
            use_tc_tiling_on_sc=False, needs_layout_passes=False),
        scratch_types=[
            pltpu.VMEM((2, DCH, SLAB, W), jnp.float32),
            pltpu.VMEM((SLAB, W), jnp.float32),
            pltpu.VMEM((SLAB, W), jnp.float32),
            pltpu.VMEM((SLAB, W), jnp.float32),
            pltpu.VMEM((SLAB, W), jnp.float32),
            pltpu.VMEM((SLAB, WP), jnp.float32),
            pltpu.VMEM((2, 9, 4, WO), jnp.float32),
            pltpu.VMEM((2, 4, WO), jnp.float32),
            pltpu.SemaphoreType.DMA,
            pltpu.SemaphoreType.DMA,
            pltpu.SemaphoreType.DMA,
            pltpu.SemaphoreType.DMA,
            pltpu.SemaphoreType.DMA,
            pltpu.SemaphoreType.DMA,
        ],
    )(_sc_body)


@jax.jit
def kernel(cost, spg):
    return _sc_regression()(cost.reshape(B, D, H, W), spg)


# speedup vs baseline: 5.3111x; 1.6014x over previous
import functools

import jax
import jax.numpy as jnp
from jax import lax
from jax.experimental import pallas as pl
from jax.experimental.pallas import tpu as pltpu
from jax.experimental.pallas import tpu_sc as plsc

B, D, H, W = 4, 48, 128, 256
UP = 4
HO, WO = H * UP, W * UP
BANDS = 8
RPB = H // BANDS
TROWS = RPB + 2
CROWS = RPB + 16
DCH = 3
NCH = D // DCH
ORPB = RPB * UP
NG = ORPB // 8
WQ = WO // 4
L = 16


def _top2_update(v, m1, m2, i1, i2, dscaled):
    dv = jnp.full((L,), dscaled, jnp.float32)
    c1 = v > m1
    c2 = v > m2
    m2n = jnp.where(c1, m1, jnp.where(c2, v, m2))
    i2n = jnp.where(c1, i1, jnp.where(c2, dv, i2))
    m1n = jnp.where(c1, v, m1)
    i1n = jnp.where(c1, dv, i1)
    return m1n, m2n, i1n, i2n


def _sc_body(cost_hbm, spg_hbm, out_hbm,
             costbuf, m1s, m2s, i1s, i2s, d4p, spgbuf, outbuf,
             sem_c0, sem_c1, sem_s0, sem_s1, sem_o0, sem_o1):
    cid = lax.axis_index("c")
    sid = lax.axis_index("s")
    wid = cid * 16 + sid
    b = wid // BANDS
    band = wid % BANDS
    y0 = band * RPB
    rbase = pl.multiple_of(jnp.clip(y0 - 8, 0, H - CROWS), 8)
    goff = (y0 - 1) - rbase
    ybase = band * ORPB

    sem_c = (sem_c0, sem_c1)
    sem_s = (sem_s0, sem_s1)
    sem_o = (sem_o0, sem_o1)

    def cost_copy(ci, par):
        return pltpu.make_async_copy(
            cost_hbm.at[b, pl.ds(ci * DCH, DCH), pl.ds(rbase, CROWS), :],
            costbuf.at[par], sem_c[par])

    def spg_copy(g, xq, par):
        return pltpu.make_async_copy(
            spg_hbm.at[b, :, pl.ds(pl.multiple_of(ybase + 8 * g, 8), 8),
                       pl.ds(xq * WQ, WQ)],
            spgbuf.at[par], sem_s[par])

    def out_copy(g, xq, par):
        return pltpu.make_async_copy(
            outbuf.at[par],
            out_hbm.at[b, pl.ds(pl.multiple_of(ybase + 8 * g, 8), 8),
                       pl.ds(xq * WQ, WQ)],
            sem_o[par])

    cost_copy(0, 0).start()
    spg_copy(jnp.int32(0), 0, 0).start()
    spg_copy(jnp.int32(0), 1, 1).start()

    neg = jnp.full((L,), -jnp.inf, jnp.float32)
    zero = jnp.zeros((L,), jnp.float32)

    @pl.loop(0, TROWS)
    def _(r):
        for jj in range(W // L):
            sl = pl.ds(jj * L, L)
            m1s[r, sl] = neg
            m2s[r, sl] = neg
            i1s[r, sl] = zero
            i2s[r, sl] = zero

    for ci in range(NCH):
        par = ci & 1
        if ci + 1 < NCH:
            cost_copy(ci + 1, 1 - par).start()
        cost_copy(ci, par).wait()

        @pl.loop(0, TROWS)
        def _(r, ci=ci, par=par):
            sr = jnp.clip(r + goff, 0, CROWS - 1)

            @pl.loop(0, W // (2 * L))
            def _(jj, r=r, sr=sr, ci=ci, par=par):
                for half in range(2):
                    sl = pl.ds(jj * L + half * (W // 2), L)
                    m1 = m1s[r, sl]
                    m2 = m2s[r, sl]
                    i1 = i1s[r, sl]
                    i2 = i2s[r, sl]
                    for dd in range(DCH):
                        v = costbuf[par, dd, sr, sl]
                        m1, m2, i1, i2 = _top2_update(
                            v, m1, m2, i1, i2, float(4 * (ci * DCH + dd)))
                    m1s[r, sl] = m1
                    m2s[r, sl] = m2
                    i1s[r, sl] = i1
                    i2s[r, sl] = i2

    @pl.loop(0, TROWS)
    def _(r):
        grow = y0 - 1 + r
        validf = ((grow >= 0) & (grow < H)).astype(jnp.float32)
        for jj in range(W // L):
            sl = pl.ds(jj * L, L)
            m1 = m1s[r, sl]
            m2 = m2s[r, sl]
            i1 = i1s[r, sl]
            i2 = i2s[r, sl]
            e = jnp.exp(m2 - m1)
            w1 = 1.0 / (1.0 + e)
            d4p[r, sl] = (i2 + (i1 - i2) * w1) * validf

    q = lax.iota(jnp.int32, L) >> 2
    zi = jnp.zeros((L,), jnp.int32)

    @pl.loop(0, NG)
    def _(g):
        rv = [zi + (2 * g + m) for m in range(4)]
        for xq in range(4):
            p = xq & 1
            spg_copy(g, xq, p).wait()

            if xq >= 2:
                out_copy(jnp.int32(0), 0, p).wait()
            else:
                @pl.when(g > 0)
                def _(p=p):
                    out_copy(jnp.int32(0), 0, p).wait()

            @pl.loop(0, WQ // L)
            def _(j, p=p, xq=xq, rv=rv):
                cb = q + (xq * (WQ // 4) + 4 * j - 1)
                taps = []
                for m in range(4):
                    row = []
                    for kx in range(3):
                        cv = cb + kx
                        if xq == 0 and kx == 0:
                            t = plsc.load_gather(d4p, [rv[m], jnp.maximum(cv, 0)])
                            t = jnp.where(cv >= 0, t, 0.0)
                        elif xq == 3 and kx == 2:
                            t = plsc.load_gather(
                                d4p, [rv[m], jnp.minimum(cv, W - 1)])
                            t = jnp.where(cv <= W - 1, t, 0.0)
                        else:
                            t = plsc.load_gather(d4p, [rv[m], cv])
                        row.append(t)
                    taps.append(row)
                for r8 in range(8):
                    lyi = r8 // 4
                    acc = None
                    for ky in range(3):
                        for kx in range(3):
                            term = (taps[lyi + ky][kx] *
                                    spgbuf[p, 3 * ky + kx, r8, pl.ds(j * L, L)])
                            acc = term if acc is None else acc + term
                    outbuf[p, r8, pl.ds(j * L, L)] = acc

            out_copy(g, xq, p).start()
            if xq < 2:
                spg_copy(g, xq + 2, p).start()
            else:
                spg_copy(jnp.minimum(g + 1, NG - 1), xq - 2, p).start()

    for p in range(2):
        spg_copy(jnp.int32(0), 0, p).wait()
        out_copy(jnp.int32(0), 0, p).wait()


@functools.cache
def _sc_regression():
    return functools.partial(
        pl.kernel,
        out_type=jax.ShapeDtypeStruct((B, HO, WO), jnp.float32),
        mesh=plsc.VectorSubcoreMesh(core_axis_name="c", subcore_axis_name="s"),
        compiler_params=pltpu.CompilerParams(needs_layout_passes=False),
        scratch_types=[
            pltpu.VMEM((2, DCH, CROWS, W), jnp.float32),
            pltpu.VMEM((TROWS, W), jnp.float32),
            pltpu.VMEM((TROWS, W), jnp.float32),
            pltpu.VMEM((TROWS, W), jnp.float32),
            pltpu.VMEM((TROWS, W), jnp.float32),
            pltpu.VMEM((TROWS, W), jnp.float32),
            pltpu.VMEM((2, 9, 8, WQ), jnp.float32),
            pltpu.VMEM((2, 8, WQ), jnp.float32),
            pltpu.SemaphoreType.DMA,
            pltpu.SemaphoreType.DMA,
            pltpu.SemaphoreType.DMA,
            pltpu.SemaphoreType.DMA,
            pltpu.SemaphoreType.DMA,
            pltpu.SemaphoreType.DMA,
        ],
    )(_sc_body)


@jax.jit
def kernel(cost, spg):
    return _sc_regression()(cost.reshape(B, D, H, W), spg)
